# Initial kernel scaffold; baseline (speedup 1.0000x reference)
import jax
import jax.numpy as jnp
from jax.experimental import pallas as pl


def kernel(x):
    raise NotImplementedError("write your pallas kernel here")


# speedup vs baseline: 57.1577x; 57.1577x over previous
import functools

import jax
import jax.numpy as jnp
from jax.experimental import pallas as pl
from jax.experimental.pallas import tpu as pltpu

K = 16


def _sort_desc_16(p):
    n = len(p)
    k = 2
    while k <= n:
        j = k // 2
        while j >= 1:
            for i in range(n):
                l = i ^ j
                if l > i:
                    if (i & k) == 0:
                        hi = jnp.maximum(p[i], p[l])
                        lo = jnp.minimum(p[i], p[l])
                    else:
                        hi = jnp.minimum(p[i], p[l])
                        lo = jnp.maximum(p[i], p[l])
                    p[i], p[l] = hi, lo
            j //= 2
        k *= 2
    return p


def _bitonic_merge_desc_16(p):
    n = len(p)
    j = n // 2
    while j >= 1:
        for base in range(0, n, 2 * j):
            for i in range(base, base + j):
                hi = jnp.maximum(p[i], p[i + j])
                lo = jnp.minimum(p[i], p[i + j])
                p[i], p[i + j] = hi, lo
        j //= 2
    return p


def _topk_body(x_ref, o_ref):
    s = x_ref.shape[1]
    g = s // K
    p = [x_ref[0, j * g:(j + 1) * g, :] for j in range(K)]
    p = _sort_desc_16(p)
    while g > 1:
        h = g // 2
        a = [q[:h] for q in p]
        b = [q[h:] for q in p]
        p = [jnp.maximum(a[j], b[K - 1 - j]) for j in range(K)]
        p = _bitonic_merge_desc_16(p)
        g = h
    o_ref[0] = jnp.concatenate(p, axis=0)


@jax.jit
def kernel(x):
    b, s, f = x.shape
    return pl.pallas_call(
        _topk_body,
        grid=(b,),
        in_specs=[pl.BlockSpec((1, s, f), lambda i: (i, 0, 0))],
        out_specs=pl.BlockSpec((1, K, f), lambda i: (i, 0, 0)),
        out_shape=jax.ShapeDtypeStruct((b, K, f), x.dtype),
        compiler_params=pltpu.CompilerParams(
            dimension_semantics=("arbitrary",),
        ),
    )(x)


# speedup vs baseline: 127.4577x; 2.2299x over previous
import functools

import jax
import jax.numpy as jnp
from jax.experimental import pallas as pl
from jax.experimental.pallas import tpu as pltpu

K = 16
STRIP = 8 * K


def _oddeven_merge(lo, hi, r):
    step = r * 2
    if step < hi - lo:
        yield from _oddeven_merge(lo, hi, step)
        yield from _oddeven_merge(lo + r, hi, step)
        yield from ((i, i + r) for i in range(lo + r, hi - r, step))
    else:
        yield (lo, lo + r)


def _oddeven_sort_pairs(lo, hi):
    if hi - lo > 1:
        mid = lo + (hi - lo) // 2
        yield from _oddeven_sort_pairs(lo, mid)
        yield from _oddeven_sort_pairs(mid, hi)
        yield from _oddeven_merge(lo, hi - 1, 1)


_SORT16 = tuple(_oddeven_sort_pairs(0, K))


def _ce(p, i, j):
    hi = jnp.maximum(p[i], p[j])
    lo = jnp.minimum(p[i], p[j])
    p[i], p[j] = hi, lo


def _sort_desc(p):
    for i, j in _SORT16:
        _ce(p, i, j)
    return p


def _bitonic_merge_desc(p):
    j = K // 2
    while j >= 1:
        for base in range(0, K, 2 * j):
            for i in range(base, base + j):
                _ce(p, i, i + j)
        j //= 2
    return p


def _merge_sorted(acc, new):
    p = [jnp.maximum(acc[j], new[K - 1 - j]) for j in range(K)]
    return _bitonic_merge_desc(p)


def _strip_planes(v):
    return _sort_desc([v[j * 8:(j + 1) * 8] for j in range(K)])


def _topk_body(x_ref, o_ref):
    s = x_ref.shape[1]
    nstrips = s // STRIP

    acc = _strip_planes(x_ref[0, 0:STRIP, :])

    def body(t, acc):
        v = x_ref[0, pl.ds(t * STRIP, STRIP), :]
        return tuple(_merge_sorted(list(acc), _strip_planes(v)))

    acc = list(jax.lax.fori_loop(1, nstrips, body, tuple(acc)))

    g = 8
    while g > 1:
        h = g // 2
        a = [q[:h] for q in acc]
        b = [q[h:] for q in acc]
        acc = _bitonic_merge_desc(
            [jnp.maximum(a[j], b[K - 1 - j]) for j in range(K)])
        g = h

    o_ref[0] = jnp.concatenate(acc, axis=0)


@jax.jit
def kernel(x):
    b, s, f = x.shape
    return pl.pallas_call(
        _topk_body,
        grid=(b,),
        in_specs=[pl.BlockSpec((1, s, f), lambda i: (i, 0, 0))],
        out_specs=pl.BlockSpec((1, K, f), lambda i: (i, 0, 0)),
        out_shape=jax.ShapeDtypeStruct((b, K, f), x.dtype),
        compiler_params=pltpu.CompilerParams(
            dimension_semantics=("arbitrary",),
        ),
    )(x)


# speedup vs baseline: 132.2350x; 1.0375x over previous
import functools

import jax
import jax.numpy as jnp
from jax.experimental import pallas as pl
from jax.experimental.pallas import tpu as pltpu

K = 16
STRIP = 8 * K


def _oddeven_merge(lo, hi, r):
    step = r * 2
    if step < hi - lo:
        yield from _oddeven_merge(lo, hi, step)
        yield from _oddeven_merge(lo + r, hi, step)
        yield from ((i, i + r) for i in range(lo + r, hi - r, step))
    else:
        yield (lo, lo + r)


def _oddeven_sort_pairs(lo, hi):
    if hi - lo > 1:
        mid = lo + (hi - lo) // 2
        yield from _oddeven_sort_pairs(lo, mid)
        yield from _oddeven_sort_pairs(mid, hi)
        yield from _oddeven_merge(lo, hi - 1, 1)


_SORT16 = tuple(_oddeven_sort_pairs(0, K))


def _ce(p, i, j):
    hi = jnp.maximum(p[i], p[j])
    lo = jnp.minimum(p[i], p[j])
    p[i], p[j] = hi, lo


def _sort_desc(p):
    for i, j in _SORT16:
        _ce(p, i, j)
    return p


def _bitonic_merge_desc(p):
    j = K // 2
    while j >= 1:
        for base in range(0, K, 2 * j):
            for i in range(base, base + j):
                _ce(p, i, i + j)
        j //= 2
    return p


def _merge_sorted(acc, new):
    p = [jnp.maximum(acc[j], new[K - 1 - j]) for j in range(K)]
    return _bitonic_merge_desc(p)


def _strip_planes(v):
    return _sort_desc([v[j * 8:(j + 1) * 8] for j in range(K)])


def _topk_body(x_ref, o_ref):
    s = x_ref.shape[1]
    npairs = s // (2 * STRIP)

    first = _strip_planes(x_ref[0, 0:STRIP, :])
    second = _strip_planes(x_ref[0, STRIP:2 * STRIP, :])
    acc = _merge_sorted(first, second)

    def body(t, acc):
        v1 = x_ref[0, pl.ds(t * (2 * STRIP), STRIP), :]
        v2 = x_ref[0, pl.ds(t * (2 * STRIP) + STRIP, STRIP), :]
        m = _merge_sorted(_strip_planes(v1), _strip_planes(v2))
        return tuple(_merge_sorted(list(acc), m))

    acc = list(jax.lax.fori_loop(1, npairs, body, tuple(acc)))

    g = 8
    while g > 1:
        h = g // 2
        a = [q[:h] for q in acc]
        b = [q[h:] for q in acc]
        acc = _bitonic_merge_desc(
            [jnp.maximum(a[j], b[K - 1 - j]) for j in range(K)])
        g = h

    o_ref[0] = jnp.concatenate(acc, axis=0)


@jax.jit
def kernel(x):
    b, s, f = x.shape
    return pl.pallas_call(
        _topk_body,
        grid=(b,),
        in_specs=[pl.BlockSpec((1, s, f), lambda i: (i, 0, 0))],
        out_specs=pl.BlockSpec((1, K, f), lambda i: (i, 0, 0)),
        out_shape=jax.ShapeDtypeStruct((b, K, f), x.dtype),
        compiler_params=pltpu.CompilerParams(
            dimension_semantics=("arbitrary",),
        ),
    )(x)


# speedup vs baseline: 138.7471x; 1.0492x over previous
import functools

import jax
import jax.numpy as jnp
from jax import lax
from jax.experimental import pallas as pl
from jax.experimental.pallas import tpu as pltpu
from jax.experimental.pallas import tpu_sc as plsc

K = 16
STRIP = 8 * K


def _oddeven_merge(lo, hi, r):
    step = r * 2
    if step < hi - lo:
        yield from _oddeven_merge(lo, hi, step)
        yield from _oddeven_merge(lo + r, hi, step)
        yield from ((i, i + r) for i in range(lo + r, hi - r, step))
    else:
        yield (lo, lo + r)


def _oddeven_sort_pairs(lo, hi):
    if hi - lo > 1:
        mid = lo + (hi - lo) // 2
        yield from _oddeven_sort_pairs(lo, mid)
        yield from _oddeven_sort_pairs(mid, hi)
        yield from _oddeven_merge(lo, hi - 1, 1)


_SORT16 = tuple(_oddeven_sort_pairs(0, K))


def _ce(p, i, j):
    hi = jnp.maximum(p[i], p[j])
    lo = jnp.minimum(p[i], p[j])
    p[i], p[j] = hi, lo


def _sort_desc(p):
    for i, j in _SORT16:
        _ce(p, i, j)
    return p


def _bitonic_merge_desc(p):
    j = K // 2
    while j >= 1:
        for base in range(0, K, 2 * j):
            for i in range(base, base + j):
                _ce(p, i, i + j)
        j //= 2
    return p


def _merge_sorted(acc, new):
    p = [jnp.maximum(acc[j], new[K - 1 - j]) for j in range(K)]
    return _bitonic_merge_desc(p)


def _strip_planes(v):
    return _sort_desc([v[j * 8:(j + 1) * 8] for j in range(K)])


def _topk_body(x_ref, o_ref):
    s = x_ref.shape[1]
    npairs = s // (2 * STRIP)

    first = _strip_planes(x_ref[0, 0:STRIP, :])
    second = _strip_planes(x_ref[0, STRIP:2 * STRIP, :])
    acc = _merge_sorted(first, second)

    def body(t, acc):
        v1 = x_ref[0, pl.ds(t * (2 * STRIP), STRIP), :]
        v2 = x_ref[0, pl.ds(t * (2 * STRIP) + STRIP, STRIP), :]
        m = _merge_sorted(_strip_planes(v1), _strip_planes(v2))
        return tuple(_merge_sorted(list(acc), m))

    acc = list(jax.lax.fori_loop(1, npairs, body, tuple(acc)))

    g = 8
    while g > 1:
        h = g // 2
        a = [q[:h] for q in acc]
        b = [q[h:] for q in acc]
        acc = _bitonic_merge_desc(
            [jnp.maximum(a[j], b[K - 1 - j]) for j in range(K)])
        g = h

    o_ref[0] = jnp.concatenate(acc, axis=0)


def _tc_topk(x, nb):
    b, s, f = x.shape
    return pl.pallas_call(
        _topk_body,
        grid=(nb,),
        in_specs=[pl.BlockSpec((1, s, f), lambda i: (i, 0, 0))],
        out_specs=pl.BlockSpec((1, K, f), lambda i: (i, 0, 0)),
        out_shape=jax.ShapeDtypeStruct((nb, K, f), x.dtype),
        compiler_params=pltpu.CompilerParams(
            dimension_semantics=("arbitrary",),
        ),
    )(x)


SC_NW = 32
SC_SBLK = 512
SC_L = 16


def _sc_process_block(buf, acc):
    def body(g, acc):
        p = _sort_desc([buf[g * K + j] for j in range(K)])
        return tuple(_merge_sorted(list(acc), p))

    return lax.fori_loop(0, SC_SBLK // K, body, acc)


def _sc_topk(x, b0, nb):
    full_b, s, f = x.shape
    nfc = f // SC_L
    units = nb * nfc
    assert units % SC_NW == 0
    nblocks = s // SC_SBLK
    mesh = plsc.VectorSubcoreMesh(core_axis_name="c", subcore_axis_name="s")

    @functools.partial(
        pl.kernel,
        mesh=mesh,
        out_type=jax.ShapeDtypeStruct((nb, K, f), jnp.float32),
        compiler_params=pltpu.CompilerParams(use_tc_tiling_on_sc=False),
        scratch_types=[
            pltpu.VMEM((SC_SBLK, SC_L), jnp.float32),
            pltpu.VMEM((SC_SBLK, SC_L), jnp.float32),
            pltpu.VMEM((K, SC_L), jnp.float32),
            pltpu.SemaphoreType.DMA,
            pltpu.SemaphoreType.DMA,
        ],
    )
    def k(x_hbm, o_hbm, buf0, buf1, obuf, sem0, sem1):
        wid = lax.axis_index("s") * 2 + lax.axis_index("c")

        for t in range(units // SC_NW):
            u = wid + SC_NW * t
            b = b0 + u // nfc
            f0 = (u % nfc) * SC_L

            def src(blk):
                return x_hbm.at[b, pl.ds(blk * SC_SBLK, SC_SBLK),
                                pl.ds(f0, SC_L)]

            pltpu.async_copy(src(0), buf0, sem0)
            pltpu.async_copy(src(1), buf1, sem1)

            acc0 = tuple(jnp.full((SC_L,), -jnp.inf, jnp.float32)
                         for _ in range(K))

            def body(i, acc, _src=src):
                pltpu.make_async_copy(_src(0), buf0, sem0).wait()
                acc = _sc_process_block(buf0, acc)

                @pl.when(2 * i + 2 < nblocks)
                def _():
                    pltpu.async_copy(_src(2 * i + 2), buf0, sem0)

                pltpu.make_async_copy(_src(1), buf1, sem1).wait()
                acc = _sc_process_block(buf1, acc)

                @pl.when(2 * i + 3 < nblocks)
                def _():
                    pltpu.async_copy(_src(2 * i + 3), buf1, sem1)

                return acc

            acc = lax.fori_loop(0, nblocks // 2, body, acc0)

            for j in range(K):
                obuf[j] = acc[j]
            pltpu.sync_copy(obuf, o_hbm.at[u // nfc, :, pl.ds(f0, SC_L)])

    return k(x)


TC_BATCHES = 48


@jax.jit
def kernel(x):
    b, s, f = x.shape
    tc_out = _tc_topk(x, TC_BATCHES)
    sc_out = _sc_topk(x, TC_BATCHES, b - TC_BATCHES)
    return jnp.concatenate([tc_out, sc_out], axis=0)
